# baseline (speedup 1.0000x reference)
import functools

import jax
import jax.numpy as jnp
from jax import lax
from jax.experimental import pallas as pl
from jax.experimental.pallas import tpu as pltpu
from jax.experimental.pallas import tpu_sc as plsc

_D = 32
_NC = 2
_NS = 16
_NW = _NC * _NS
_LANES = 16
_NTQ = 4
_NBQ = 8
_REP = 1025


def _lookup(xt, wtab, n_t, n_b):
    tq_len = n_t // _NTQ
    bw = n_b // _NBQ
    ngrp = bw // _LANES
    win = ((tq_len + 6 + 7) // 8) * 8
    mesh = plsc.VectorSubcoreMesh(core_axis_name="c", subcore_axis_name="s")

    @functools.partial(
        pl.kernel,
        mesh=mesh,
        compiler_params=pltpu.CompilerParams(needs_layout_passes=False),
        out_type=jax.ShapeDtypeStruct((n_t * _D, n_b), jnp.float32),
        scratch_types=[
            pltpu.VMEM((win, bw), jnp.int32),
            pltpu.VMEM((_D * _D,), jnp.float32),
            pltpu.VMEM((_LANES * _REP,), jnp.float32),
            pltpu.VMEM((_D, bw), jnp.float32),
            pltpu.VMEM((_D, bw), jnp.float32),
            pltpu.SemaphoreType.DMA,
            pltpu.SemaphoreType.DMA,
            pltpu.SemaphoreType.DMA,
        ],
    )
    def k(xt_hbm, wtab_hbm, out_hbm, xl_v, wtab_v, rep_v, rows_v0, rows_v1,
          xsem, osem0, osem1):
        wid = lax.axis_index("s") * _NC + lax.axis_index("c")
        tq = wid // _NBQ
        b0 = (wid % _NBQ) * bw
        t_base = tq * tq_len
        rows_v = (rows_v0, rows_v1)
        osem = (osem0, osem1)
        iota = lax.iota(jnp.int32, _LANES)
        skew = iota * _REP

        toff = lax.rem(t_base, 8)
        t_al = pl.multiple_of(t_base - toff, 8)
        pltpu.async_copy(xt_hbm.at[pl.ds(t_al, win), pl.ds(b0, bw)],
                         xl_v, xsem)
        pltpu.sync_copy(wtab_hbm, wtab_v)

        def rep_body(kk, carry):
            v = wtab_v[pl.ds(kk * _LANES, _LANES)]
            for l in range(_LANES):
                rep_v[pl.ds(l * _REP + kk * _LANES, _LANES)] = v
            return carry

        lax.fori_loop(0, (_D * _D) // _LANES, rep_body, 0)
        pltpu.make_async_copy(xt_hbm.at[pl.ds(0, win), pl.ds(0, bw)],
                              xl_v, xsem).wait()

        def pair_body(p, carry):
            for b in range(2):
                tl = p * 2 + b
                @pl.when(p > 0)
                def _():
                    pltpu.make_async_copy(
                        rows_v[b], out_hbm.at[pl.ds(0, _D), pl.ds(b0, bw)],
                        osem[b]).wait()

                @plsc.parallel_loop(0, ngrp, 1, unroll=4)
                def group_body(g):
                    cvec = xl_v[toff + tl, pl.ds(g * _LANES, _LANES)]
                    coffs = jnp.minimum(jnp.maximum(cvec, 0), _D - 1)
                    bsvec = coffs + skew
                    for d in range(_D):
                        gth = plsc.load_gather(rep_v, [bsvec + d * _D])
                        rows_v[b][d, pl.ds(g * _LANES, _LANES)] = gth

                ro = pl.multiple_of((t_base + tl) * _D, _D)
                pltpu.async_copy(
                    rows_v[b],
                    out_hbm.at[pl.ds(ro, _D), pl.ds(b0, bw)],
                    osem[b])
            return carry

        lax.fori_loop(0, tq_len // 2, pair_body, 0)
        for b in range(2):
            pltpu.make_async_copy(
                rows_v[b], out_hbm.at[pl.ds(0, _D), pl.ds(b0, bw)],
                osem[b]).wait()

    return k(xt, wtab)


def kernel(x, W):
    n_b, n_t = x.shape
    wtab = W[:_D].T.reshape(-1)
    out2 = _lookup(x.T, wtab, n_t, n_b)
    out = out2.reshape(n_t, _D, n_b).transpose(2, 0, 1)
    return out


# speedup vs baseline: 1.7121x; 1.7121x over previous
import functools

import jax
import jax.numpy as jnp
from jax import lax
from jax.experimental import pallas as pl
from jax.experimental.pallas import tpu as pltpu
from jax.experimental.pallas import tpu_sc as plsc

_D = 32
_NC = 2
_NS = 16
_NW = _NC * _NS
_LANES = 16
_BW = 128
_TCH = 4
_REP = 1025


def _lookup(xt, wtab, n_t, n_b):
    chunks = n_t // _TCH
    mesh = plsc.VectorSubcoreMesh(core_axis_name="c", subcore_axis_name="s")

    @functools.partial(
        pl.kernel,
        mesh=mesh,
        compiler_params=pltpu.CompilerParams(needs_layout_passes=False),
        out_type=jax.ShapeDtypeStruct((n_t * _D, n_b), jnp.float32),
        scratch_types=[
            pltpu.VMEM((n_t, _BW), jnp.int32),
            pltpu.VMEM((_D * _D,), jnp.float32),
            pltpu.VMEM((_LANES * _REP,), jnp.float32),
            pltpu.VMEM((_TCH * _D, _BW), jnp.float32),
            pltpu.VMEM((_TCH * _D, _BW), jnp.float32),
            pltpu.SemaphoreType.DMA,
            pltpu.SemaphoreType.DMA,
            pltpu.SemaphoreType.DMA,
        ],
    )
    def k(xt_hbm, wtab_hbm, out_hbm, xl_v, wtab_v, rep_v, rows_v0, rows_v1,
          xsem, osem0, osem1):
        wid = lax.axis_index("s") * _NC + lax.axis_index("c")
        b0 = wid * _BW
        rows_v = (rows_v0, rows_v1)
        osem = (osem0, osem1)
        iota = lax.iota(jnp.int32, _LANES)
        skew = iota * _REP

        pltpu.async_copy(xt_hbm.at[:, pl.ds(b0, _BW)], xl_v, xsem)
        pltpu.sync_copy(wtab_hbm, wtab_v)

        def rep_body(kk, carry):
            v = wtab_v[pl.ds(kk * _LANES, _LANES)]
            for l in range(_LANES):
                rep_v[pl.ds(l * _REP + kk * _LANES, _LANES)] = v
            return carry

        lax.fori_loop(0, (_D * _D) // _LANES, rep_body, 0)
        pltpu.make_async_copy(xt_hbm.at[:, pl.ds(0, _BW)], xl_v, xsem).wait()

        def pair_body(p, carry):
            for b in range(2):
                c = p * 2 + b
                t0 = c * _TCH
                @pl.when(p > 0)
                def _():
                    pltpu.make_async_copy(
                        rows_v[b], out_hbm.at[pl.ds(0, _TCH * _D),
                                              pl.ds(b0, _BW)],
                        osem[b]).wait()

                @plsc.parallel_loop(0, _TCH * (_BW // _LANES), 1, unroll=2)
                def group_body(i):
                    tl = i // (_BW // _LANES)
                    g = i % (_BW // _LANES)
                    cvec = xl_v[t0 + tl, pl.ds(g * _LANES, _LANES)]
                    coffs = jnp.minimum(jnp.maximum(cvec, 0), _D - 1)
                    bsvec = coffs + skew
                    for d in range(_D):
                        gth = plsc.load_gather(rep_v, [bsvec + d * _D])
                        rows_v[b][tl * _D + d, pl.ds(g * _LANES, _LANES)] = gth

                pltpu.async_copy(
                    rows_v[b],
                    out_hbm.at[pl.ds(t0 * _D, _TCH * _D), pl.ds(b0, _BW)],
                    osem[b])
            return carry

        lax.fori_loop(0, chunks // 2, pair_body, 0)
        for b in range(2):
            pltpu.make_async_copy(
                rows_v[b], out_hbm.at[pl.ds(0, _TCH * _D), pl.ds(b0, _BW)],
                osem[b]).wait()

    return k(xt, wtab)


def kernel(x, W):
    n_b, n_t = x.shape
    wtab = W[:_D].T.reshape(-1)
    out2 = _lookup(x.T, wtab, n_t, n_b)
    out = out2.reshape(n_t, _D, n_b).transpose(2, 0, 1)
    return out
